# baseline (speedup 1.0000x reference)
import jax
import jax.numpy as jnp
from jax.experimental import pallas as pl
from jax.experimental.pallas import tpu as pltpu


def _matmul_kernel(x_ref, w_ref, h_ref):
    h_ref[...] = jnp.dot(x_ref[...], w_ref[...],
                         preferred_element_type=jnp.float32)


def _prep_kernel(a_ref, deg_ref, a8_ref):
    i = pl.program_id(0)

    @pl.when(i == 0)
    def _():
        deg_ref[...] = jnp.zeros_like(deg_ref)

    a = a_ref[...]
    deg_ref[...] += jnp.sum(a, axis=0, keepdims=True)
    a8_ref[...] = a.astype(jnp.int8)


def _agg_kernel(bl_r, bl_c, a8_ref, h_ref, deg_ref, b_ref, out_ref, acc_ref):
    ct = pl.program_id(0)
    rt = pl.program_id(1)

    @pl.when(rt == 0)
    def _():
        acc_ref[...] = jnp.zeros_like(acc_ref)

    rows = pl.ds(rt * bl_r, bl_r)
    d_r = jax.lax.rsqrt(deg_ref[rows, :] + 1.0)
    g = (h_ref[rows, :] * d_r).astype(jnp.bfloat16)
    a = a8_ref[...].astype(jnp.bfloat16)
    acc_ref[...] += jax.lax.dot_general(
        a, g, (((0,), (0,)), ((), ())),
        preferred_element_type=jnp.float32)

    @pl.when(rt == pl.num_programs(1) - 1)
    def _():
        cols = pl.ds(ct * bl_c, bl_c)
        d_c = jax.lax.rsqrt(deg_ref[cols, :] + 1.0)
        res = (d_c * acc_ref[...] + (d_c * d_c) * h_ref[cols, :]
               + b_ref[...])
        out_ref[...] = jnp.maximum(res, 0.0)


@jax.jit
def kernel(x, edge_index, W, b):
    adj = edge_index
    n, d_in = x.shape
    d_out = W.shape[1]

    rh = min(1024, n)
    h = pl.pallas_call(
        _matmul_kernel,
        grid=(n // rh,),
        in_specs=[
            pl.BlockSpec((rh, d_in), lambda i: (i, 0)),
            pl.BlockSpec((d_in, d_out), lambda i: (0, 0)),
        ],
        out_specs=pl.BlockSpec((rh, d_out), lambda i: (i, 0)),
        out_shape=jax.ShapeDtypeStruct((n, d_out), jnp.float32),
    )(x, W)

    r1 = min(512, n)
    deg_sum, a8 = pl.pallas_call(
        _prep_kernel,
        grid=(n // r1,),
        in_specs=[pl.BlockSpec((r1, n), lambda i: (i, 0))],
        out_specs=[
            pl.BlockSpec((1, n), lambda i: (0, 0)),
            pl.BlockSpec((r1, n), lambda i: (i, 0)),
        ],
        out_shape=[
            jax.ShapeDtypeStruct((1, n), jnp.float32),
            jax.ShapeDtypeStruct((n, n), jnp.int8),
        ],
    )(adj)

    deg_t = deg_sum.reshape(n, 1)
    b2 = b.reshape(1, d_out)

    bl_r = min(1024, n)
    bl_c = min(1024, n)

    def agg_body(*refs):
        _agg_kernel(bl_r, bl_c, *refs)

    out = pl.pallas_call(
        agg_body,
        grid=(n // bl_c, n // bl_r),
        in_specs=[
            pl.BlockSpec((bl_r, bl_c), lambda ct, rt: (rt, ct)),
            pl.BlockSpec((n, d_out), lambda ct, rt: (0, 0)),
            pl.BlockSpec((n, 1), lambda ct, rt: (0, 0)),
            pl.BlockSpec((1, d_out), lambda ct, rt: (0, 0)),
        ],
        out_specs=pl.BlockSpec((bl_c, d_out), lambda ct, rt: (ct, 0)),
        out_shape=jax.ShapeDtypeStruct((n, d_out), jnp.float32),
        scratch_shapes=[pltpu.VMEM((bl_c, d_out), jnp.float32)],
        compiler_params=pltpu.CompilerParams(
            dimension_semantics=("parallel", "arbitrary"),
            fuse_transposed_lhs_in_matmul=True),
    )(a8, h, deg_t, b2)

    return out


# speedup vs baseline: 1.0703x; 1.0703x over previous
import jax
import jax.numpy as jnp
from jax.experimental import pallas as pl
from jax.experimental.pallas import tpu as pltpu


def _matmul_kernel(x_ref, w_ref, h_ref):
    h_ref[...] = jnp.dot(x_ref[...], w_ref[...],
                         preferred_element_type=jnp.float32)


def _prep_kernel(a_ref, deg_ref, a8_ref):
    i = pl.program_id(0)

    @pl.when(i == 0)
    def _():
        deg_ref[...] = jnp.zeros_like(deg_ref)

    a = a_ref[...]
    deg_ref[...] += jnp.sum(a, axis=0, keepdims=True)
    a8_ref[...] = a.astype(jnp.int8)


def _agg_kernel(bl_r, bl_c, a8_ref, h_ref, deg_ref, b_ref, out_ref, acc_ref):
    ct = pl.program_id(0)
    rt = pl.program_id(1)

    @pl.when(rt == 0)
    def _():
        acc_ref[...] = jnp.zeros_like(acc_ref)

    rows = pl.ds(rt * bl_r, bl_r)
    d_r = jax.lax.rsqrt(deg_ref[rows, :] + 1.0)
    g = (h_ref[rows, :] * d_r).astype(jnp.bfloat16)
    a = a8_ref[...].astype(jnp.bfloat16)
    acc_ref[...] += jax.lax.dot_general(
        a, g, (((0,), (0,)), ((), ())),
        preferred_element_type=jnp.float32)

    @pl.when(rt == pl.num_programs(1) - 1)
    def _():
        cols = pl.ds(ct * bl_c, bl_c)
        d_c = jax.lax.rsqrt(deg_ref[cols, :] + 1.0)
        res = (d_c * acc_ref[...] + (d_c * d_c) * h_ref[cols, :]
               + b_ref[...])
        out_ref[...] = jnp.maximum(res, 0.0)


@jax.jit
def kernel(x, edge_index, W, b):
    adj = edge_index
    n, d_in = x.shape
    d_out = W.shape[1]

    rh = min(1024, n)
    h = pl.pallas_call(
        _matmul_kernel,
        grid=(n // rh,),
        in_specs=[
            pl.BlockSpec((rh, d_in), lambda i: (i, 0)),
            pl.BlockSpec((d_in, d_out), lambda i: (0, 0)),
        ],
        out_specs=pl.BlockSpec((rh, d_out), lambda i: (i, 0)),
        out_shape=jax.ShapeDtypeStruct((n, d_out), jnp.float32),
    )(x, W)

    r1 = min(512, n)
    deg_sum, a8 = pl.pallas_call(
        _prep_kernel,
        grid=(n // r1,),
        in_specs=[pl.BlockSpec((r1, n), lambda i: (i, 0))],
        out_specs=[
            pl.BlockSpec((1, n), lambda i: (0, 0)),
            pl.BlockSpec((r1, n), lambda i: (i, 0)),
        ],
        out_shape=[
            jax.ShapeDtypeStruct((1, n), jnp.float32),
            jax.ShapeDtypeStruct((n, n), jnp.int8),
        ],
    )(adj)

    deg_t = deg_sum.reshape(n, 1)
    b2 = b.reshape(1, d_out)

    bl_r = min(1024, n)
    bl_c = min(1024, n)

    def agg_body(*refs):
        _agg_kernel(bl_r, bl_c, *refs)

    out = pl.pallas_call(
        agg_body,
        grid=(n // bl_c, n // bl_r),
        in_specs=[
            pl.BlockSpec((bl_r, bl_c), lambda ct, rt: (rt, ct)),
            pl.BlockSpec((n, d_out), lambda ct, rt: (0, 0)),
            pl.BlockSpec((n, 1), lambda ct, rt: (0, 0)),
            pl.BlockSpec((1, d_out), lambda ct, rt: (0, 0)),
        ],
        out_specs=pl.BlockSpec((bl_c, d_out), lambda ct, rt: (ct, 0)),
        out_shape=jax.ShapeDtypeStruct((n, d_out), jnp.float32),
        scratch_shapes=[pltpu.VMEM((bl_c, d_out), jnp.float32)],
        compiler_params=pltpu.CompilerParams(
            dimension_semantics=("parallel", "arbitrary")),
    )(a8, h, deg_t, b2)

    return out
